# Initial kernel scaffold; baseline (speedup 1.0000x reference)
import jax
import jax.numpy as jnp
from jax.experimental import pallas as pl


def kernel(w_ih_f, w_hh_f, b_f, w_ih_b, w_hh_b, b_b, w_pos, b_pos, w_biaff, root, x00, x01, x02, x03, x04, x05, x06, x07, x08, x09, x10, x11, x12, x13, x14, x15, x16, x17, x18, x19, x20, x21, x22, x23, x24, x25, x26, x27, x28, x29, x30, x31):
    raise NotImplementedError("write your pallas kernel here")


# speedup vs baseline: 1.4482x; 1.4482x over previous
import jax
import jax.numpy as jnp
from jax.experimental import pallas as pl
from jax.experimental.pallas import tpu as pltpu


def _round_up(x, m):
    return ((x + m - 1) // m) * m


def _fuse_gates(wf, wb, hd):
    k = wf.shape[0]
    return jnp.concatenate([wf.reshape(k, 4, hd), wb.reshape(k, 4, hd)],
                           axis=2).reshape(k, 8 * hd)


def _bilstm_kernel(x_ref, wih_ref, whh_ref, b_ref, head_ref, pos_ref, dep_ref,
                   gx_sc, embf_sc, embb_sc):
    T, Bc, E = x_ref.shape
    G8 = wih_ref.shape[1]
    HID = whh_ref.shape[0]
    Hd = HID // 2
    N = T
    POS_PAD = pos_ref.shape[2]
    DEP_PAD = dep_ref.shape[2]

    xx = x_ref[...].reshape(T * Bc, E)
    gx_sc[...] = jnp.dot(xx, wih_ref[...],
                         preferred_element_type=jnp.float32) + b_ref[...]

    lane = jax.lax.broadcasted_iota(jnp.int32, (Bc, G8), 1)
    is_g = (lane >= 2 * HID) & (lane < 3 * HID)
    is_fwd = jax.lax.rem(lane, 2 * Hd) < Hd

    h = jnp.zeros((Bc, HID), jnp.float32)
    c = jnp.zeros((Bc, HID), jnp.float32)
    for t in range(T):
        gf = gx_sc[t * Bc:(t + 1) * Bc, :]
        gb = gx_sc[(T - 1 - t) * Bc:(T - t) * Bc, :]
        gates = jnp.where(is_fwd, gf, gb) + jnp.dot(
            h, whh_ref[...], preferred_element_type=jnp.float32)
        th = jnp.tanh(jnp.where(is_g, gates, 0.5 * gates))
        act = jnp.where(is_g, th, 0.5 * th + 0.5)
        ii = act[:, 0 * HID:1 * HID]
        ff = act[:, 1 * HID:2 * HID]
        gg = act[:, 2 * HID:3 * HID]
        oo = act[:, 3 * HID:4 * HID]
        c = ff * c + ii * gg
        h = oo * jnp.tanh(c)
        embf_sc[t] = h[:, :Hd]
        embb_sc[T - 1 - t] = h[:, Hd:]

    whead = head_ref[0:HID, :]
    bhead = head_ref[HID:HID + 1, :]
    root = head_ref[HID + 1:HID + 2, 0:HID]

    embf = embf_sc[...]
    embb = embb_sc[...]
    embs_list = [jnp.concatenate([embf[:, b, :], embb[:, b, :]], axis=-1)
                 for b in range(Bc)]
    embs_2d = jnp.concatenate(embs_list, axis=0)

    big = jnp.dot(embs_2d, whead,
                  preferred_element_type=jnp.float32) + bhead

    pad_n = DEP_PAD - (N + 1)
    zero_rows = (jnp.zeros((pad_n, HID), jnp.float32) if pad_n > 0 else None)
    dn_t = (((1,), (1,)), ((), ()))

    for b in range(Bc):
        r0, r1 = b * N, (b + 1) * N
        pos_ref[b] = big[r0:r1, 0:POS_PAD]
        tmp_b = big[r0:r1, POS_PAD:POS_PAD + HID]
        colb_b = big[r0:r1, POS_PAD + HID:POS_PAD + HID + 1]
        parts = [root, embs_list[b]] + ([zero_rows] if pad_n > 0 else [])
        heads_b = jnp.concatenate(parts, axis=0)
        s = jax.lax.dot_general(tmp_b, heads_b, dn_t,
                                preferred_element_type=jnp.float32)
        dep_ref[b] = s + colb_b


@jax.jit
def _forward(params, xs):
    x = jnp.stack(xs, axis=0).astype(jnp.float32)
    B, N, E = x.shape
    T = N
    Hd = params["w_hh_f"].shape[0]
    HID = 2 * Hd
    NPOS = params["w_pos"].shape[1]
    G8 = 8 * Hd

    NC = 2
    Bc = B // NC

    POS_PAD = _round_up(NPOS, 128)
    DEP_PAD = _round_up(N + 1, 128)
    HEAD_W = POS_PAD + _round_up(HID + 1, 128)

    x_tm = jnp.transpose(x, (1, 0, 2)).astype(jnp.bfloat16)

    wih = _fuse_gates(params["w_ih_f"], params["w_ih_b"], Hd).astype(
        jnp.bfloat16)
    bias = _fuse_gates(params["b_f"], params["b_b"], Hd)
    zH = jnp.zeros((Hd, 4 * Hd), jnp.float32)
    whh = jnp.concatenate([_fuse_gates(params["w_hh_f"], zH, Hd),
                           _fuse_gates(zH, params["w_hh_b"], Hd)],
                          axis=0)

    wb_full = params["w_biaff"]
    w11 = wb_full[:HID, :HID]
    w1b = wb_full[:HID, HID:HID + 1]
    wb1 = wb_full[HID:HID + 1, :HID]
    wbb = wb_full[HID:HID + 1, HID:HID + 1]
    whead = jnp.zeros((HID, HEAD_W), jnp.float32)
    whead = whead.at[:, :NPOS].set(params["w_pos"])
    whead = whead.at[:, POS_PAD:POS_PAD + HID].set(w11)
    whead = whead.at[:, POS_PAD + HID:POS_PAD + HID + 1].set(w1b)
    bhead = jnp.zeros((1, HEAD_W), jnp.float32)
    bhead = bhead.at[:, :NPOS].set(params["b_pos"])
    bhead = bhead.at[:, POS_PAD:POS_PAD + HID].set(wb1)
    bhead = bhead.at[:, POS_PAD + HID:POS_PAD + HID + 1].set(wbb)
    rootrow = jnp.zeros((1, HEAD_W), jnp.float32).at[:, :HID].set(
        params["root"].reshape(1, HID))
    head_slab = jnp.concatenate([whead, bhead, rootrow], axis=0)

    pos_pad, dep_pad = pl.pallas_call(
        _bilstm_kernel,
        grid=(NC,),
        in_specs=[
            pl.BlockSpec((T, Bc, E), lambda i: (0, i, 0)),
            pl.BlockSpec((E, G8), lambda i: (0, 0)),
            pl.BlockSpec((HID, G8), lambda i: (0, 0)),
            pl.BlockSpec((1, G8), lambda i: (0, 0)),
            pl.BlockSpec((HID + 2, HEAD_W), lambda i: (0, 0)),
        ],
        out_specs=(
            pl.BlockSpec((Bc, N, POS_PAD), lambda i: (i, 0, 0)),
            pl.BlockSpec((Bc, N, DEP_PAD), lambda i: (i, 0, 0)),
        ),
        out_shape=(jax.ShapeDtypeStruct((B, N, POS_PAD), jnp.float32),
                   jax.ShapeDtypeStruct((B, N, DEP_PAD), jnp.float32)),
        scratch_shapes=[pltpu.VMEM((T * Bc, G8), jnp.float32),
                        pltpu.VMEM((N, Bc, Hd), jnp.float32),
                        pltpu.VMEM((N, Bc, Hd), jnp.float32)],
        compiler_params=pltpu.CompilerParams(
            dimension_semantics=("parallel",),
            vmem_limit_bytes=48 * 1024 * 1024),
    )(x_tm, wih, whh, bias, head_slab)

    return pos_pad[:, :, :NPOS], dep_pad[:, :, :N + 1]


def kernel(w_ih_f, w_hh_f, b_f, w_ih_b, w_hh_b, b_b, w_pos, b_pos, w_biaff,
           root, x00, x01, x02, x03, x04, x05, x06, x07, x08, x09, x10, x11,
           x12, x13, x14, x15, x16, x17, x18, x19, x20, x21, x22, x23, x24,
           x25, x26, x27, x28, x29, x30, x31):
    params = {
        "w_ih_f": w_ih_f, "w_hh_f": w_hh_f, "b_f": b_f,
        "w_ih_b": w_ih_b, "w_hh_b": w_hh_b, "b_b": b_b,
        "w_pos": w_pos, "b_pos": b_pos, "w_biaff": w_biaff, "root": root,
    }
    xs = [x00, x01, x02, x03, x04, x05, x06, x07, x08, x09,
          x10, x11, x12, x13, x14, x15, x16, x17, x18, x19,
          x20, x21, x22, x23, x24, x25, x26, x27, x28, x29,
          x30, x31]
    return _forward(params, xs)


# speedup vs baseline: 3.4023x; 2.3493x over previous
import jax
import jax.numpy as jnp
from jax.experimental import pallas as pl
from jax.experimental.pallas import tpu as pltpu


def _round_up(x, m):
    return ((x + m - 1) // m) * m


def _make_kernel(T, Bc, E, Hd, NPOS):
    HID = 2 * Hd
    G8 = 8 * Hd
    N = T
    Bh = Bc // 2
    DEP_PAD = _round_up(N + 1, 128)
    POS_PAD = _round_up(NPOS, 128)

    def body(*refs):
        x_refs = refs[:2 * Bc]
        wih_ref, whh_ref, b_ref, head_ref, pos_ref, dep_ref = refs[2 * Bc:2 * Bc + 6]
        xtm, gx_sc, embf_sc, embb_sc, sem = refs[2 * Bc + 6:]

        i = pl.program_id(0)

        @pl.when(i == 0)
        def _():
            for j in range(Bc):
                pltpu.make_async_copy(x_refs[j], xtm.at[:, j], sem).start()
            for j in range(Bc):
                pltpu.make_async_copy(x_refs[j], xtm.at[:, j], sem).wait()

        @pl.when(i == 1)
        def _():
            for j in range(Bc):
                pltpu.make_async_copy(x_refs[Bc + j], xtm.at[:, j], sem).start()
            for j in range(Bc):
                pltpu.make_async_copy(x_refs[Bc + j], xtm.at[:, j], sem).wait()

        xx = xtm[...].reshape(T * Bc, E)
        gx_sc[...] = jnp.dot(xx, wih_ref[...],
                             preferred_element_type=jnp.float32) + b_ref[...]

        lane = jax.lax.broadcasted_iota(jnp.int32, (Bh, G8), 1)
        r = jax.lax.rem(lane, 4 * Hd)
        is_g = (r >= 2 * Hd) & (r < 3 * Hd)
        is_fwd = lane < 4 * Hd

        def gate(act, g):
            return jnp.concatenate(
                [act[:, g * Hd:(g + 1) * Hd],
                 act[:, (4 + g) * Hd:(5 + g) * Hd]], axis=-1)

        def step(h, c, gf, gb):
            gates = jnp.where(is_fwd, gf, gb) + jnp.dot(
                h, whh_ref[...], preferred_element_type=jnp.float32)
            th = jnp.tanh(jnp.where(is_g, gates, 0.5 * gates))
            act = jnp.where(is_g, th, 0.5 * th + 0.5)
            c = gate(act, 1) * c + gate(act, 0) * gate(act, 2)
            h = gate(act, 3) * jnp.tanh(c)
            return h, c

        h1 = jnp.zeros((Bh, HID), jnp.float32)
        c1 = jnp.zeros((Bh, HID), jnp.float32)
        h2 = jnp.zeros((Bh, HID), jnp.float32)
        c2 = jnp.zeros((Bh, HID), jnp.float32)
        for t in range(T):
            bf = t * Bc
            bb = (T - 1 - t) * Bc
            h1, c1 = step(h1, c1, gx_sc[bf:bf + Bh], gx_sc[bb:bb + Bh])
            h2, c2 = step(h2, c2, gx_sc[bf + Bh:bf + Bc], gx_sc[bb + Bh:bb + Bc])
            embf_sc[t, 0:Bh] = h1[:, :Hd]
            embf_sc[t, Bh:Bc] = h2[:, :Hd]
            embb_sc[T - 1 - t, 0:Bh] = h1[:, Hd:]
            embb_sc[T - 1 - t, Bh:Bc] = h2[:, Hd:]

        whead = head_ref[0:HID, :]
        bhead = head_ref[HID:HID + 1, :]
        root = head_ref[HID + 1:HID + 2, 0:HID]

        embf = embf_sc[...]
        embb = embb_sc[...]
        embs_list = [jnp.concatenate([embf[:, b, :], embb[:, b, :]], axis=-1)
                     for b in range(Bc)]
        embs_2d = jnp.concatenate(embs_list, axis=0)

        big = jnp.dot(embs_2d, whead,
                      preferred_element_type=jnp.float32) + bhead

        pad_n = DEP_PAD - (N + 1)
        zero_rows = (jnp.zeros((pad_n, HID), jnp.float32)
                     if pad_n > 0 else None)
        dn_t = (((1,), (1,)), ((), ()))

        for b in range(Bc):
            r0, r1 = b * N, (b + 1) * N
            pos_ref[b] = big[r0:r1, 0:NPOS]
            tmp_b = big[r0:r1, POS_PAD:POS_PAD + HID]
            colb_b = big[r0:r1, POS_PAD + HID:POS_PAD + HID + 1]
            parts = [root, embs_list[b]] + ([zero_rows] if pad_n > 0 else [])
            heads_b = jnp.concatenate(parts, axis=0)
            s = jax.lax.dot_general(tmp_b, heads_b, dn_t,
                                    preferred_element_type=jnp.float32)
            dep_ref[b] = (s + colb_b)[:, :N + 1]

    return body


@jax.jit
def _forward(params, xs):
    B = len(xs)
    N, E = xs[0].shape
    T = N
    Hd = params["w_hh_f"].shape[0]
    HID = 2 * Hd
    NPOS = params["w_pos"].shape[1]
    G8 = 8 * Hd

    NC = 2
    Bc = B // NC

    POS_PAD = _round_up(NPOS, 128)
    HEAD_W = POS_PAD + _round_up(HID + 1, 128)

    f32 = jnp.float32
    wih = jnp.concatenate([params["w_ih_f"], params["w_ih_b"]],
                          axis=1).astype(f32)
    bias = jnp.concatenate([params["b_f"], params["b_b"]],
                           axis=1).astype(f32)
    z = jnp.zeros((Hd, 4 * Hd), f32)
    whh = jnp.concatenate(
        [jnp.concatenate([params["w_hh_f"], z], axis=1),
         jnp.concatenate([z, params["w_hh_b"]], axis=1)], axis=0)

    wb_full = params["w_biaff"]
    w11 = wb_full[:HID, :HID]
    w1b = wb_full[:HID, HID:HID + 1]
    wb1 = wb_full[HID:HID + 1, :HID]
    wbb = wb_full[HID:HID + 1, HID:HID + 1]
    whead = jnp.zeros((HID, HEAD_W), f32)
    whead = whead.at[:, :NPOS].set(params["w_pos"])
    whead = whead.at[:, POS_PAD:POS_PAD + HID].set(w11)
    whead = whead.at[:, POS_PAD + HID:POS_PAD + HID + 1].set(w1b)
    bhead = jnp.zeros((1, HEAD_W), f32)
    bhead = bhead.at[:, :NPOS].set(params["b_pos"])
    bhead = bhead.at[:, POS_PAD:POS_PAD + HID].set(wb1)
    bhead = bhead.at[:, POS_PAD + HID:POS_PAD + HID + 1].set(wbb)
    rootrow = jnp.zeros((1, HEAD_W), f32).at[:, :HID].set(
        params["root"].reshape(1, HID))
    head_slab = jnp.concatenate([whead, bhead, rootrow], axis=0)

    any_spec = pl.BlockSpec(memory_space=pl.ANY)
    pos, dep = pl.pallas_call(
        _make_kernel(T, Bc, E, Hd, NPOS),
        grid=(NC,),
        in_specs=[any_spec] * B + [
            pl.BlockSpec((E, G8), lambda i: (0, 0)),
            pl.BlockSpec((HID, G8), lambda i: (0, 0)),
            pl.BlockSpec((1, G8), lambda i: (0, 0)),
            pl.BlockSpec((HID + 2, HEAD_W), lambda i: (0, 0)),
        ],
        out_specs=(
            pl.BlockSpec((Bc, N, NPOS), lambda i: (i, 0, 0)),
            pl.BlockSpec((Bc, N, N + 1), lambda i: (i, 0, 0)),
        ),
        out_shape=(jax.ShapeDtypeStruct((B, N, NPOS), f32),
                   jax.ShapeDtypeStruct((B, N, N + 1), f32)),
        scratch_shapes=[pltpu.VMEM((T, Bc, E), f32),
                        pltpu.VMEM((T * Bc, G8), f32),
                        pltpu.VMEM((N, Bc, Hd), f32),
                        pltpu.VMEM((N, Bc, Hd), f32),
                        pltpu.SemaphoreType.DMA],
        compiler_params=pltpu.CompilerParams(
            dimension_semantics=("parallel",),
            vmem_limit_bytes=48 * 1024 * 1024),
    )(*xs, wih, whh, bias, head_slab)

    return pos, dep


def kernel(w_ih_f, w_hh_f, b_f, w_ih_b, w_hh_b, b_b, w_pos, b_pos, w_biaff,
           root, x00, x01, x02, x03, x04, x05, x06, x07, x08, x09, x10, x11,
           x12, x13, x14, x15, x16, x17, x18, x19, x20, x21, x22, x23, x24,
           x25, x26, x27, x28, x29, x30, x31):
    params = {
        "w_ih_f": w_ih_f, "w_hh_f": w_hh_f, "b_f": b_f,
        "w_ih_b": w_ih_b, "w_hh_b": w_hh_b, "b_b": b_b,
        "w_pos": w_pos, "b_pos": b_pos, "w_biaff": w_biaff, "root": root,
    }
    xs = [x00, x01, x02, x03, x04, x05, x06, x07, x08, x09,
          x10, x11, x12, x13, x14, x15, x16, x17, x18, x19,
          x20, x21, x22, x23, x24, x25, x26, x27, x28, x29,
          x30, x31]
    return _forward(params, xs)
